# baseline (speedup 1.0000x reference)
import functools

import numpy as np

import jax
import jax.numpy as jnp
from jax import lax
from jax.experimental import pallas as pl
from jax.experimental.pallas import tpu as pltpu
from jax.experimental.pallas import tpu_sc as plsc

EPS = 1e-07
N_DATA = 100000
FEAT = 128
BATCH = 128
K = 1024
NCE_T = 0.07
NCE_M = 0.5
MPN = float(K) / float(N_DATA)

NC, NS = 2, 16
NW = NC * NS
NEG = BATCH * K
PER_TILE = NEG // NW
CH = 128
NCHUNK = PER_TILE // CH

_PERM = np.array([32 * (p // 32) + (p % 32) // 2 + 16 * ((p % 32) % 2)
                  for p in range(FEAT)], dtype=np.int32)


def _pack_rows(buf, pk):
    def row(r, carry):
        for c in range(4):
            a = buf[r, pl.ds(32 * c, 16)]
            b = buf[r, pl.ds(32 * c + 16, 16)]
            pk[r, pl.ds(32 * c, 32)] = plsc.pack(
                a, b, format=plsc.PackFormat.INTERLEAVED)
        return carry

    lax.fori_loop(0, CH, row, 0)


def _gather_body(mem_s, mem_t, cidx, wsn, wtn,
                 idx_all, bs0, bs1, bt0, bt1, ps0, ps1, pt0, pt1,
                 gs0, gs1, gt0, gt1, ws0, ws1, wt0, wt1):
    wid = lax.axis_index("s") * NC + lax.axis_index("c")
    base = pl.multiple_of(wid * PER_TILE, PER_TILE)
    pltpu.sync_copy(cidx.at[pl.ds(base, PER_TILE)], idx_all)
    bufs = ((bs0, bt0, ps0, pt0, gs0, gt0, ws0, wt0),
            (bs1, bt1, ps1, pt1, gs1, gt1, ws1, wt1))

    def idx_slice(i):
        return idx_all.at[pl.ds(pl.multiple_of(i * CH, CH), CH)]

    for b in range(2):
        bs, bt = bufs[b][0], bufs[b][1]
        gs, gt = bufs[b][4], bufs[b][5]
        pltpu.async_copy(mem_s.at[idx_slice(b)], bs, gs)
        pltpu.async_copy(mem_t.at[idx_slice(b)], bt, gt)

    def pair(i2, carry):
        for b in range(2):
            i = i2 * 2 + b
            bs, bt, pks, pkt, gs, gt, ws, wt = bufs[b]
            off = pl.multiple_of(base + i * CH, CH)
            pltpu.make_async_copy(mem_s.at[idx_slice(i)], bs, gs).wait()
            pltpu.make_async_copy(mem_t.at[idx_slice(i)], bt, gt).wait()

            @pl.when(i >= 2)
            def _():
                pltpu.make_async_copy(pks, wsn.at[pl.ds(off, CH)], ws).wait()
                pltpu.make_async_copy(pkt, wtn.at[pl.ds(off, CH)], wt).wait()

            _pack_rows(bs, pks)
            _pack_rows(bt, pkt)
            pltpu.async_copy(pks, wsn.at[pl.ds(off, CH)], ws)
            pltpu.async_copy(pkt, wtn.at[pl.ds(off, CH)], wt)

            @pl.when(i + 2 < NCHUNK)
            def _():
                pltpu.async_copy(mem_s.at[idx_slice(i + 2)], bs, gs)
                pltpu.async_copy(mem_t.at[idx_slice(i + 2)], bt, gt)
        return carry

    lax.fori_loop(0, NCHUNK // 2, pair, 0)

    for b in range(2):
        pks, pkt = bufs[b][2], bufs[b][3]
        ws, wt = bufs[b][6], bufs[b][7]
        i = NCHUNK - 2 + b
        off = pl.multiple_of(base + i * CH, CH)
        pltpu.make_async_copy(pks, wsn.at[pl.ds(off, CH)], ws).wait()
        pltpu.make_async_copy(pkt, wtn.at[pl.ds(off, CH)], wt).wait()


def _gather_pos_body(mem_s, mem_t, pidx, wsp, wtp,
                     idx_v, buf_s, buf_t, sem_s, sem_t):
    wid = lax.axis_index("s") * NC + lax.axis_index("c")

    @pl.when(wid == 0)
    def _():
        pltpu.sync_copy(pidx, idx_v)
        c1 = pltpu.async_copy(mem_s.at[idx_v], buf_s, sem_s)
        c2 = pltpu.async_copy(mem_t.at[idx_v], buf_t, sem_t)
        c1.wait()
        c2.wait()
        pltpu.sync_copy(buf_s, wsp)
        pltpu.sync_copy(buf_t, wtp)


def _sc_mesh():
    return plsc.VectorSubcoreMesh(
        core_axis_name="c", subcore_axis_name="s",
        num_cores=NC, num_subcores=NS)


@functools.cache
def _make_gather():
    return pl.kernel(
        _gather_body,
        out_type=(
            jax.ShapeDtypeStruct((NEG, FEAT), jnp.bfloat16),
            jax.ShapeDtypeStruct((NEG, FEAT), jnp.bfloat16),
        ),
        mesh=_sc_mesh(),
        compiler_params=pltpu.CompilerParams(needs_layout_passes=False),
        scratch_types=[pltpu.VMEM((PER_TILE,), jnp.int32)]
        + [pltpu.VMEM((CH, FEAT), jnp.float32)] * 4
        + [pltpu.VMEM((CH, FEAT), jnp.bfloat16)] * 4
        + [pltpu.SemaphoreType.DMA] * 8,
    )


@functools.cache
def _make_gather_pos():
    return pl.kernel(
        _gather_pos_body,
        out_type=(
            jax.ShapeDtypeStruct((BATCH, FEAT), jnp.float32),
            jax.ShapeDtypeStruct((BATCH, FEAT), jnp.float32),
        ),
        mesh=_sc_mesh(),
        scratch_types=[
            pltpu.VMEM((BATCH,), jnp.int32),
            pltpu.VMEM((BATCH, FEAT), jnp.float32),
            pltpu.VMEM((BATCH, FEAT), jnp.float32),
            pltpu.SemaphoreType.DMA,
            pltpu.SemaphoreType.DMA,
        ],
    )

_DN = (((1,), (1,)), ((), ()))


def _l2rows(x):
    return x * lax.rsqrt(jnp.sum(x * x, axis=-1, keepdims=True))


def _c1_body(f_s, f_t, W_s, b_s, W_t, b_t, wsp, wtp, W_mt, b_mt, idx2,
             h_s_o, h_t_o, upd_s_o, upd_t_o, s1_o):
    hs = _l2rows(lax.dot_general(f_s[...], W_s[...], _DN,
                                 preferred_element_type=jnp.float32) + b_s[...])
    ht = _l2rows(lax.dot_general(f_t[...], W_t[...], _DN,
                                 preferred_element_type=jnp.float32) + b_t[...])
    h_s_o[...] = hs
    h_t_o[...] = ht
    et0 = _l2rows(lax.dot_general(ht * wsp[...], W_mt[...], _DN,
                                  preferred_element_type=jnp.float32) + b_mt[...])
    es0 = _l2rows(lax.dot_general(hs * wtp[...], W_mt[...], _DN,
                                  preferred_element_type=jnp.float32) + b_mt[...])
    p = jnp.exp(jnp.sum(et0 * es0, axis=1) / NCE_T) / float(N_DATA)
    s1_o[0, 0] = jnp.sum(jnp.log(p / (p + MPN + EPS)))
    upd_s = _l2rows(NCE_M * wsp[...] + (1.0 - NCE_M) * hs)
    upd_t = _l2rows(NCE_M * wtp[...] + (1.0 - NCE_M) * ht)
    iv = idx2[0, :]
    eq = iv[:, None] == iv[None, :]
    bi = lax.broadcasted_iota(jnp.int32, (BATCH, BATCH), 1)
    win = jnp.max(jnp.where(eq, bi, -1), axis=1)
    oh = (bi == win[:, None]).astype(jnp.float32)
    pick = (((1,), (0,)), ((), ()))
    upd_s_o[...] = lax.dot_general(oh, upd_s, pick,
                                   precision=lax.Precision.HIGHEST,
                                   preferred_element_type=jnp.float32)
    upd_t_o[...] = lax.dot_general(oh, upd_t, pick,
                                   precision=lax.Precision.HIGHEST,
                                   preferred_element_type=jnp.float32)


_c1 = pl.pallas_call(
    _c1_body,
    out_shape=(
        jax.ShapeDtypeStruct((BATCH, FEAT), jnp.float32),
        jax.ShapeDtypeStruct((BATCH, FEAT), jnp.float32),
        jax.ShapeDtypeStruct((BATCH, FEAT), jnp.float32),
        jax.ShapeDtypeStruct((BATCH, FEAT), jnp.float32),
        jax.ShapeDtypeStruct((1, 1), jnp.float32),
    ),
    out_specs=(
        pl.BlockSpec((BATCH, FEAT), lambda: (0, 0)),
        pl.BlockSpec((BATCH, FEAT), lambda: (0, 0)),
        pl.BlockSpec((BATCH, FEAT), lambda: (0, 0)),
        pl.BlockSpec((BATCH, FEAT), lambda: (0, 0)),
        pl.BlockSpec((1, 1), lambda: (0, 0), memory_space=pltpu.SMEM),
    ),
)


NSLAB = 125
SLAB = N_DATA // NSLAB


def _c2_body(wsn, wtn, hs, ht, W_mt, b_mt, mt_in, ms_in,
             s0_o, mt_out, ms_out):
    i = pl.program_id(0)
    zt = lax.dot_general(wsn[0] * ht[0], W_mt[...], _DN,
                         preferred_element_type=jnp.float32) + b_mt[...]
    zs = lax.dot_general(wtn[0] * hs[0], W_mt[...], _DN,
                         preferred_element_type=jnp.float32) + b_mt[...]
    dot = jnp.sum(zt * zs, axis=1)
    nt = jnp.sum(zt * zt, axis=1)
    ns = jnp.sum(zs * zs, axis=1)
    s = dot * lax.rsqrt(nt * ns)
    p = jnp.exp(s / NCE_T) / float(N_DATA)
    part = jnp.sum(jnp.log(MPN / (p + MPN + EPS)))

    @pl.when(i == 0)
    def _():
        s0_o[0, 0] = 0.0

    s0_o[0, 0] += part
    mt_out[...] = mt_in[...]
    ms_out[...] = ms_in[...]


def _slab_map(i):
    j = jnp.minimum(i, NSLAB - 1)
    return (j, 0, 0)


_c2 = pl.pallas_call(
    _c2_body,
    grid=(BATCH,),
    in_specs=[
        pl.BlockSpec((1, K, FEAT), lambda i: (i, 0, 0)),
        pl.BlockSpec((1, K, FEAT), lambda i: (i, 0, 0)),
        pl.BlockSpec((1, 1, FEAT), lambda i: (i, 0, 0)),
        pl.BlockSpec((1, 1, FEAT), lambda i: (i, 0, 0)),
        pl.BlockSpec((FEAT, FEAT), lambda i: (0, 0)),
        pl.BlockSpec((1, FEAT), lambda i: (0, 0)),
        pl.BlockSpec((1, SLAB, FEAT), _slab_map),
        pl.BlockSpec((1, SLAB, FEAT), _slab_map),
    ],
    out_specs=(
        pl.BlockSpec((1, 1), lambda i: (0, 0), memory_space=pltpu.SMEM),
        pl.BlockSpec((1, SLAB, FEAT), _slab_map),
        pl.BlockSpec((1, SLAB, FEAT), _slab_map),
    ),
    out_shape=(
        jax.ShapeDtypeStruct((1, 1), jnp.float32),
        jax.ShapeDtypeStruct((NSLAB, SLAB, FEAT), jnp.float32),
        jax.ShapeDtypeStruct((NSLAB, SLAB, FEAT), jnp.float32),
    ),
)


def _scat_body(idx_ref, upd_t, upd_s, mtc, msc, out_t, out_s, sem):
    def fire(b, carry):
        r = idx_ref[b]
        pltpu.async_copy(upd_t.at[b], out_t.at[r], sem)
        pltpu.async_copy(upd_s.at[b], out_s.at[r], sem)
        return carry

    lax.fori_loop(0, BATCH, fire, 0)

    def drain(b, carry):
        pltpu.make_async_copy(upd_t.at[0], out_t.at[0], sem).wait()
        pltpu.make_async_copy(upd_s.at[0], out_s.at[0], sem).wait()
        return carry

    lax.fori_loop(0, BATCH, drain, 0)


def _scatter2(mtc, msc, idx, upd_t, upd_s):
    grid_spec = pltpu.PrefetchScalarGridSpec(
        num_scalar_prefetch=1,
        grid=(1,),
        in_specs=[
            pl.BlockSpec((BATCH, FEAT), lambda i, idx: (0, 0)),
            pl.BlockSpec((BATCH, FEAT), lambda i, idx: (0, 0)),
            pl.BlockSpec(memory_space=pl.ANY),
            pl.BlockSpec(memory_space=pl.ANY),
        ],
        out_specs=(
            pl.BlockSpec(memory_space=pl.ANY),
            pl.BlockSpec(memory_space=pl.ANY),
        ),
        scratch_shapes=[pltpu.SemaphoreType.DMA],
    )
    return pl.pallas_call(
        _scat_body,
        grid_spec=grid_spec,
        out_shape=(
            jax.ShapeDtypeStruct((N_DATA, FEAT), jnp.float32),
            jax.ShapeDtypeStruct((N_DATA, FEAT), jnp.float32),
        ),
        input_output_aliases={3: 0, 4: 1},
    )(idx, upd_t, upd_s, mtc, msc)


def kernel(f_s, f_t, idx, contrast_idx, W_s, b_s, W_t, b_t,
           memory_s, memory_t, W_mt, b_mt):
    cidx = contrast_idx.reshape(NEG)
    wsp, wtp = _make_gather_pos()(memory_s, memory_t, idx)
    wsn_p, wtn_p = _make_gather()(memory_s, memory_t, cidx)
    wsn = wsn_p.reshape(BATCH, K, FEAT)
    wtn = wtn_p.reshape(BATCH, K, FEAT)
    b_s2 = b_s.reshape(1, FEAT)
    b_t2 = b_t.reshape(1, FEAT)
    b_mt2 = b_mt.reshape(1, FEAT)
    hs, ht, upd_s, upd_t, s1 = _c1(f_s, f_t, W_s, b_s2, W_t, b_t2,
                                   wsp, wtp, W_mt, b_mt2,
                                   idx.reshape(1, BATCH))
    hs_b = hs[:, _PERM].astype(jnp.bfloat16).reshape(BATCH, 1, FEAT)
    ht_b = ht[:, _PERM].astype(jnp.bfloat16).reshape(BATCH, 1, FEAT)
    W_mt_b = W_mt[:, _PERM].astype(jnp.bfloat16)
    s0, mt_copy, ms_copy = _c2(
        wsn, wtn, hs_b, ht_b, W_mt_b, b_mt2,
        memory_t.reshape(NSLAB, SLAB, FEAT), memory_s.reshape(NSLAB, SLAB, FEAT))
    loss = (-(s1[0, 0] + s0[0, 0]) / BATCH).reshape(1)
    new_mt, new_ms = _scatter2(mt_copy.reshape(N_DATA, FEAT),
                               ms_copy.reshape(N_DATA, FEAT),
                               idx, upd_t, upd_s)
    return loss, new_mt, new_ms


# speedup vs baseline: 1.1634x; 1.1634x over previous
import functools

import jax
import jax.numpy as jnp
from jax import lax
from jax.experimental import pallas as pl
from jax.experimental.pallas import tpu as pltpu
from jax.experimental.pallas import tpu_sc as plsc

EPS = 1e-07
N_DATA = 100000
FEAT = 128
BATCH = 128
K = 1024
NCE_T = 0.07
NCE_M = 0.5
MPN = float(K) / float(N_DATA)

NC, NS = 2, 16
NW = NC * NS
NEG = BATCH * K
PER_TILE = NEG // NW
CH = 128
NCHUNK = PER_TILE // CH


def _gather_body(mem_s, mem_t, cidx, wsn, wtn,
                 idx_all, bs0, bs1, bt0, bt1,
                 gs0, gs1, gt0, gt1, ws0, ws1, wt0, wt1):
    wid = lax.axis_index("s") * NC + lax.axis_index("c")
    base = pl.multiple_of(wid * PER_TILE, PER_TILE)
    pltpu.sync_copy(cidx.at[pl.ds(base, PER_TILE)], idx_all)
    bufs = ((bs0, bt0, gs0, gt0, ws0, wt0), (bs1, bt1, gs1, gt1, ws1, wt1))

    def idx_slice(i):
        return idx_all.at[pl.ds(pl.multiple_of(i * CH, CH), CH)]

    for b in range(2):
        bs, bt, gs, gt, _, _ = bufs[b]
        pltpu.async_copy(mem_s.at[idx_slice(b)], bs, gs)
        pltpu.async_copy(mem_t.at[idx_slice(b)], bt, gt)

    def pair(i2, carry):
        for b in range(2):
            i = i2 * 2 + b
            bs, bt, gs, gt, ws, wt = bufs[b]
            off = pl.multiple_of(base + i * CH, CH)
            pltpu.make_async_copy(mem_s.at[idx_slice(i)], bs, gs).wait()
            pltpu.make_async_copy(mem_t.at[idx_slice(i)], bt, gt).wait()
            pltpu.async_copy(bs, wsn.at[pl.ds(off, CH)], ws)
            pltpu.async_copy(bt, wtn.at[pl.ds(off, CH)], wt)

            @pl.when(i + 2 < NCHUNK)
            def _():
                pltpu.make_async_copy(bs, wsn.at[pl.ds(off, CH)], ws).wait()
                pltpu.make_async_copy(bt, wtn.at[pl.ds(off, CH)], wt).wait()
                pltpu.async_copy(mem_s.at[idx_slice(i + 2)], bs, gs)
                pltpu.async_copy(mem_t.at[idx_slice(i + 2)], bt, gt)
        return carry

    lax.fori_loop(0, NCHUNK // 2, pair, 0)

    for b in range(2):
        bs, bt, _, _, ws, wt = bufs[b]
        i = NCHUNK - 2 + b
        off = pl.multiple_of(base + i * CH, CH)
        pltpu.make_async_copy(bs, wsn.at[pl.ds(off, CH)], ws).wait()
        pltpu.make_async_copy(bt, wtn.at[pl.ds(off, CH)], wt).wait()


def _gather_pos_body(mem_s, mem_t, pidx, wsp, wtp,
                     idx_v, buf_s, buf_t, sem_s, sem_t):
    wid = lax.axis_index("s") * NC + lax.axis_index("c")

    @pl.when(wid == 0)
    def _():
        pltpu.sync_copy(pidx, idx_v)
        c1 = pltpu.async_copy(mem_s.at[idx_v], buf_s, sem_s)
        c2 = pltpu.async_copy(mem_t.at[idx_v], buf_t, sem_t)
        c1.wait()
        c2.wait()
        pltpu.sync_copy(buf_s, wsp)
        pltpu.sync_copy(buf_t, wtp)


def _sc_mesh():
    return plsc.VectorSubcoreMesh(
        core_axis_name="c", subcore_axis_name="s",
        num_cores=NC, num_subcores=NS)


@functools.cache
def _make_gather():
    return pl.kernel(
        _gather_body,
        out_type=(
            jax.ShapeDtypeStruct((NEG, FEAT), jnp.float32),
            jax.ShapeDtypeStruct((NEG, FEAT), jnp.float32),
        ),
        mesh=_sc_mesh(),
        scratch_types=[pltpu.VMEM((PER_TILE,), jnp.int32)]
        + [pltpu.VMEM((CH, FEAT), jnp.float32)] * 4
        + [pltpu.SemaphoreType.DMA] * 8,
    )


@functools.cache
def _make_gather_pos():
    return pl.kernel(
        _gather_pos_body,
        out_type=(
            jax.ShapeDtypeStruct((BATCH, FEAT), jnp.float32),
            jax.ShapeDtypeStruct((BATCH, FEAT), jnp.float32),
        ),
        mesh=_sc_mesh(),
        scratch_types=[
            pltpu.VMEM((BATCH,), jnp.int32),
            pltpu.VMEM((BATCH, FEAT), jnp.float32),
            pltpu.VMEM((BATCH, FEAT), jnp.float32),
            pltpu.SemaphoreType.DMA,
            pltpu.SemaphoreType.DMA,
        ],
    )

_DN = (((1,), (1,)), ((), ()))


def _l2rows(x):
    return x * lax.rsqrt(jnp.sum(x * x, axis=-1, keepdims=True))


def _c1_body(f_s, f_t, W_s, b_s, W_t, b_t, wsp, wtp, W_mt, b_mt, idx2,
             h_s_o, h_t_o, upd_s_o, upd_t_o, s1_o):
    hs = _l2rows(lax.dot_general(f_s[...], W_s[...], _DN,
                                 preferred_element_type=jnp.float32) + b_s[...])
    ht = _l2rows(lax.dot_general(f_t[...], W_t[...], _DN,
                                 preferred_element_type=jnp.float32) + b_t[...])
    h_s_o[...] = hs
    h_t_o[...] = ht
    et0 = _l2rows(lax.dot_general(ht * wsp[...], W_mt[...], _DN,
                                  preferred_element_type=jnp.float32) + b_mt[...])
    es0 = _l2rows(lax.dot_general(hs * wtp[...], W_mt[...], _DN,
                                  preferred_element_type=jnp.float32) + b_mt[...])
    p = jnp.exp(jnp.sum(et0 * es0, axis=1) / NCE_T) / float(N_DATA)
    s1_o[0, 0] = jnp.sum(jnp.log(p / (p + MPN + EPS)))
    upd_s = _l2rows(NCE_M * wsp[...] + (1.0 - NCE_M) * hs)
    upd_t = _l2rows(NCE_M * wtp[...] + (1.0 - NCE_M) * ht)
    iv = idx2[0, :]
    eq = iv[:, None] == iv[None, :]
    bi = lax.broadcasted_iota(jnp.int32, (BATCH, BATCH), 1)
    win = jnp.max(jnp.where(eq, bi, -1), axis=1)
    oh = (bi == win[:, None]).astype(jnp.float32)
    pick = (((1,), (0,)), ((), ()))
    upd_s_o[...] = lax.dot_general(oh, upd_s, pick,
                                   preferred_element_type=jnp.float32)
    upd_t_o[...] = lax.dot_general(oh, upd_t, pick,
                                   preferred_element_type=jnp.float32)


_c1 = pl.pallas_call(
    _c1_body,
    out_shape=(
        jax.ShapeDtypeStruct((BATCH, FEAT), jnp.float32),
        jax.ShapeDtypeStruct((BATCH, FEAT), jnp.float32),
        jax.ShapeDtypeStruct((BATCH, FEAT), jnp.float32),
        jax.ShapeDtypeStruct((BATCH, FEAT), jnp.float32),
        jax.ShapeDtypeStruct((1, 1), jnp.float32),
    ),
    out_specs=(
        pl.BlockSpec((BATCH, FEAT), lambda: (0, 0)),
        pl.BlockSpec((BATCH, FEAT), lambda: (0, 0)),
        pl.BlockSpec((BATCH, FEAT), lambda: (0, 0)),
        pl.BlockSpec((BATCH, FEAT), lambda: (0, 0)),
        pl.BlockSpec((1, 1), lambda: (0, 0), memory_space=pltpu.SMEM),
    ),
)


NSLAB = 125
SLAB = N_DATA // NSLAB


def _c2_body(wsn, wtn, hs, ht, W_mt, b_mt, mt_in, ms_in,
             s0_o, mt_out, ms_out):
    i = pl.program_id(0)
    zt = lax.dot_general(wsn[0] * ht[0], W_mt[...], _DN,
                         preferred_element_type=jnp.float32) + b_mt[...]
    zs = lax.dot_general(wtn[0] * hs[0], W_mt[...], _DN,
                         preferred_element_type=jnp.float32) + b_mt[...]
    dot = jnp.sum(zt * zs, axis=1)
    nt = jnp.sum(zt * zt, axis=1)
    ns = jnp.sum(zs * zs, axis=1)
    s = dot * lax.rsqrt(nt * ns)
    p = jnp.exp(s / NCE_T) / float(N_DATA)
    part = jnp.sum(jnp.log(MPN / (p + MPN + EPS)))

    @pl.when(i == 0)
    def _():
        s0_o[0, 0] = 0.0

    s0_o[0, 0] += part
    mt_out[...] = mt_in[...]
    ms_out[...] = ms_in[...]


def _slab_map(i):
    j = jnp.minimum(i, NSLAB - 1)
    return (j, 0, 0)


_c2 = pl.pallas_call(
    _c2_body,
    grid=(BATCH,),
    in_specs=[
        pl.BlockSpec((1, K, FEAT), lambda i: (i, 0, 0)),
        pl.BlockSpec((1, K, FEAT), lambda i: (i, 0, 0)),
        pl.BlockSpec((1, 1, FEAT), lambda i: (i, 0, 0)),
        pl.BlockSpec((1, 1, FEAT), lambda i: (i, 0, 0)),
        pl.BlockSpec((FEAT, FEAT), lambda i: (0, 0)),
        pl.BlockSpec((1, FEAT), lambda i: (0, 0)),
        pl.BlockSpec((1, SLAB, FEAT), _slab_map),
        pl.BlockSpec((1, SLAB, FEAT), _slab_map),
    ],
    out_specs=(
        pl.BlockSpec((1, 1), lambda i: (0, 0), memory_space=pltpu.SMEM),
        pl.BlockSpec((1, SLAB, FEAT), _slab_map),
        pl.BlockSpec((1, SLAB, FEAT), _slab_map),
    ),
    out_shape=(
        jax.ShapeDtypeStruct((1, 1), jnp.float32),
        jax.ShapeDtypeStruct((NSLAB, SLAB, FEAT), jnp.float32),
        jax.ShapeDtypeStruct((NSLAB, SLAB, FEAT), jnp.float32),
    ),
)


def _scat_body(idx_ref, upd_t, upd_s, mtc, msc, out_t, out_s, sem):
    def fire(b, carry):
        r = idx_ref[b]
        pltpu.async_copy(upd_t.at[b], out_t.at[r], sem)
        pltpu.async_copy(upd_s.at[b], out_s.at[r], sem)
        return carry

    lax.fori_loop(0, BATCH, fire, 0)

    def drain(b, carry):
        pltpu.make_async_copy(upd_t.at[0], out_t.at[0], sem).wait()
        pltpu.make_async_copy(upd_s.at[0], out_s.at[0], sem).wait()
        return carry

    lax.fori_loop(0, BATCH, drain, 0)


def _scatter2(mtc, msc, idx, upd_t, upd_s):
    grid_spec = pltpu.PrefetchScalarGridSpec(
        num_scalar_prefetch=1,
        grid=(1,),
        in_specs=[
            pl.BlockSpec((BATCH, FEAT), lambda i, idx: (0, 0)),
            pl.BlockSpec((BATCH, FEAT), lambda i, idx: (0, 0)),
            pl.BlockSpec(memory_space=pl.ANY),
            pl.BlockSpec(memory_space=pl.ANY),
        ],
        out_specs=(
            pl.BlockSpec(memory_space=pl.ANY),
            pl.BlockSpec(memory_space=pl.ANY),
        ),
        scratch_shapes=[pltpu.SemaphoreType.DMA],
    )
    return pl.pallas_call(
        _scat_body,
        grid_spec=grid_spec,
        out_shape=(
            jax.ShapeDtypeStruct((N_DATA, FEAT), jnp.float32),
            jax.ShapeDtypeStruct((N_DATA, FEAT), jnp.float32),
        ),
        input_output_aliases={3: 0, 4: 1},
    )(idx, upd_t, upd_s, mtc, msc)


def kernel(f_s, f_t, idx, contrast_idx, W_s, b_s, W_t, b_t,
           memory_s, memory_t, W_mt, b_mt):
    cidx = contrast_idx.reshape(NEG)
    wsp, wtp = _make_gather_pos()(memory_s, memory_t, idx)
    wsn, wtn = _make_gather()(memory_s, memory_t, cidx)
    b_s2 = b_s.reshape(1, FEAT)
    b_t2 = b_t.reshape(1, FEAT)
    b_mt2 = b_mt.reshape(1, FEAT)
    hs, ht, upd_s, upd_t, s1 = _c1(f_s, f_t, W_s, b_s2, W_t, b_t2,
                                   wsp, wtp, W_mt, b_mt2,
                                   idx.reshape(1, BATCH))
    s0, mt_copy, ms_copy = _c2(
        wsn.reshape(BATCH, K, FEAT), wtn.reshape(BATCH, K, FEAT),
        hs.reshape(BATCH, 1, FEAT), ht.reshape(BATCH, 1, FEAT),
        W_mt, b_mt2,
        memory_t.reshape(NSLAB, SLAB, FEAT), memory_s.reshape(NSLAB, SLAB, FEAT))
    loss = (-(s1[0, 0] + s0[0, 0]) / BATCH).reshape(1)
    new_mt, new_ms = _scatter2(mt_copy.reshape(N_DATA, FEAT),
                               ms_copy.reshape(N_DATA, FEAT),
                               idx, upd_t, upd_s)
    return loss, new_mt, new_ms


# speedup vs baseline: 1.1648x; 1.0013x over previous
import functools

import jax
import jax.numpy as jnp
from jax import lax
from jax.experimental import pallas as pl
from jax.experimental.pallas import tpu as pltpu
from jax.experimental.pallas import tpu_sc as plsc

EPS = 1e-07
N_DATA = 100000
FEAT = 128
BATCH = 128
K = 1024
NCE_T = 0.07
NCE_M = 0.5
MPN = float(K) / float(N_DATA)

NC, NS = 2, 16
NW = NC * NS
NEG = BATCH * K
PER_TILE = NEG // NW
CH = 128
NCHUNK = PER_TILE // CH


def _gather_body(mem_s, mem_t, cidx, wsn, wtn,
                 idx_all, bs0, bs1, bt0, bt1,
                 gs0, gs1, gt0, gt1, ws0, ws1, wt0, wt1):
    wid = lax.axis_index("s") * NC + lax.axis_index("c")
    base = pl.multiple_of(wid * PER_TILE, PER_TILE)
    pltpu.sync_copy(cidx.at[pl.ds(base, PER_TILE)], idx_all)
    bufs = ((bs0, bt0, gs0, gt0, ws0, wt0), (bs1, bt1, gs1, gt1, ws1, wt1))

    def idx_slice(i):
        return idx_all.at[pl.ds(pl.multiple_of(i * CH, CH), CH)]

    for b in range(2):
        bs, bt, gs, gt, _, _ = bufs[b]
        pltpu.async_copy(mem_s.at[idx_slice(b)], bs, gs)
        pltpu.async_copy(mem_t.at[idx_slice(b)], bt, gt)

    def pair(i2, carry):
        for b in range(2):
            i = i2 * 2 + b
            bs, bt, gs, gt, ws, wt = bufs[b]
            off = pl.multiple_of(base + i * CH, CH)
            pltpu.make_async_copy(mem_s.at[idx_slice(i)], bs, gs).wait()
            pltpu.make_async_copy(mem_t.at[idx_slice(i)], bt, gt).wait()
            pltpu.async_copy(bs, wsn.at[pl.ds(off, CH)], ws)
            pltpu.async_copy(bt, wtn.at[pl.ds(off, CH)], wt)

            @pl.when(i + 2 < NCHUNK)
            def _():
                pltpu.make_async_copy(bs, wsn.at[pl.ds(off, CH)], ws).wait()
                pltpu.make_async_copy(bt, wtn.at[pl.ds(off, CH)], wt).wait()
                pltpu.async_copy(mem_s.at[idx_slice(i + 2)], bs, gs)
                pltpu.async_copy(mem_t.at[idx_slice(i + 2)], bt, gt)
        return carry

    lax.fori_loop(0, NCHUNK // 2, pair, 0)

    for b in range(2):
        bs, bt, _, _, ws, wt = bufs[b]
        i = NCHUNK - 2 + b
        off = pl.multiple_of(base + i * CH, CH)
        pltpu.make_async_copy(bs, wsn.at[pl.ds(off, CH)], ws).wait()
        pltpu.make_async_copy(bt, wtn.at[pl.ds(off, CH)], wt).wait()


def _gather_pos_body(mem_s, mem_t, pidx, wsp, wtp,
                     idx_v, buf_s, buf_t, sem_s, sem_t):
    wid = lax.axis_index("s") * NC + lax.axis_index("c")

    @pl.when(wid == 0)
    def _():
        pltpu.sync_copy(pidx, idx_v)
        c1 = pltpu.async_copy(mem_s.at[idx_v], buf_s, sem_s)
        c2 = pltpu.async_copy(mem_t.at[idx_v], buf_t, sem_t)
        c1.wait()
        c2.wait()
        pltpu.sync_copy(buf_s, wsp)
        pltpu.sync_copy(buf_t, wtp)


def _sc_mesh():
    return plsc.VectorSubcoreMesh(
        core_axis_name="c", subcore_axis_name="s",
        num_cores=NC, num_subcores=NS)


@functools.cache
def _make_gather():
    return pl.kernel(
        _gather_body,
        out_type=(
            jax.ShapeDtypeStruct((NEG, FEAT), jnp.float32),
            jax.ShapeDtypeStruct((NEG, FEAT), jnp.float32),
        ),
        mesh=_sc_mesh(),
        scratch_types=[pltpu.VMEM((PER_TILE,), jnp.int32)]
        + [pltpu.VMEM((CH, FEAT), jnp.float32)] * 4
        + [pltpu.SemaphoreType.DMA] * 8,
    )


@functools.cache
def _make_gather_pos():
    return pl.kernel(
        _gather_pos_body,
        out_type=(
            jax.ShapeDtypeStruct((BATCH, FEAT), jnp.float32),
            jax.ShapeDtypeStruct((BATCH, FEAT), jnp.float32),
        ),
        mesh=_sc_mesh(),
        scratch_types=[
            pltpu.VMEM((BATCH,), jnp.int32),
            pltpu.VMEM((BATCH, FEAT), jnp.float32),
            pltpu.VMEM((BATCH, FEAT), jnp.float32),
            pltpu.SemaphoreType.DMA,
            pltpu.SemaphoreType.DMA,
        ],
    )

_DN = (((1,), (1,)), ((), ()))


def _l2rows(x):
    return x * lax.rsqrt(jnp.sum(x * x, axis=-1, keepdims=True))


def _c1_body(f_s, f_t, W_s, b_s, W_t, b_t, wsp, wtp, W_mt, b_mt, idx2,
             h_s_o, h_t_o, upd_s_o, upd_t_o, s1_o):
    hs = _l2rows(lax.dot_general(f_s[...], W_s[...], _DN,
                                 preferred_element_type=jnp.float32) + b_s[...])
    ht = _l2rows(lax.dot_general(f_t[...], W_t[...], _DN,
                                 preferred_element_type=jnp.float32) + b_t[...])
    h_s_o[...] = hs
    h_t_o[...] = ht
    et0 = _l2rows(lax.dot_general(ht * wsp[...], W_mt[...], _DN,
                                  preferred_element_type=jnp.float32) + b_mt[...])
    es0 = _l2rows(lax.dot_general(hs * wtp[...], W_mt[...], _DN,
                                  preferred_element_type=jnp.float32) + b_mt[...])
    p = jnp.exp(jnp.sum(et0 * es0, axis=1) / NCE_T) / float(N_DATA)
    s1_o[0, 0] = jnp.sum(jnp.log(p / (p + MPN + EPS)))
    upd_s = _l2rows(NCE_M * wsp[...] + (1.0 - NCE_M) * hs)
    upd_t = _l2rows(NCE_M * wtp[...] + (1.0 - NCE_M) * ht)
    iv = idx2[0, :]
    eq = iv[:, None] == iv[None, :]
    bi = lax.broadcasted_iota(jnp.int32, (BATCH, BATCH), 1)
    win = jnp.max(jnp.where(eq, bi, -1), axis=1)
    oh = (bi == win[:, None]).astype(jnp.float32)
    pick = (((1,), (0,)), ((), ()))
    upd_s_o[...] = lax.dot_general(oh, upd_s, pick,
                                   precision=lax.Precision.HIGHEST,
                                   preferred_element_type=jnp.float32)
    upd_t_o[...] = lax.dot_general(oh, upd_t, pick,
                                   precision=lax.Precision.HIGHEST,
                                   preferred_element_type=jnp.float32)


_c1 = pl.pallas_call(
    _c1_body,
    out_shape=(
        jax.ShapeDtypeStruct((BATCH, FEAT), jnp.float32),
        jax.ShapeDtypeStruct((BATCH, FEAT), jnp.float32),
        jax.ShapeDtypeStruct((BATCH, FEAT), jnp.float32),
        jax.ShapeDtypeStruct((BATCH, FEAT), jnp.float32),
        jax.ShapeDtypeStruct((1, 1), jnp.float32),
    ),
    out_specs=(
        pl.BlockSpec((BATCH, FEAT), lambda: (0, 0)),
        pl.BlockSpec((BATCH, FEAT), lambda: (0, 0)),
        pl.BlockSpec((BATCH, FEAT), lambda: (0, 0)),
        pl.BlockSpec((BATCH, FEAT), lambda: (0, 0)),
        pl.BlockSpec((1, 1), lambda: (0, 0), memory_space=pltpu.SMEM),
    ),
)


NSLAB = 125
SLAB = N_DATA // NSLAB


def _c2_body(wsn, wtn, hs, ht, W_mt, b_mt, mt_in, ms_in,
             s0_o, mt_out, ms_out):
    i = pl.program_id(0)
    zt = lax.dot_general(wsn[0] * ht[0], W_mt[...], _DN,
                         preferred_element_type=jnp.float32) + b_mt[...]
    zs = lax.dot_general(wtn[0] * hs[0], W_mt[...], _DN,
                         preferred_element_type=jnp.float32) + b_mt[...]
    dot = jnp.sum(zt * zs, axis=1)
    nt = jnp.sum(zt * zt, axis=1)
    ns = jnp.sum(zs * zs, axis=1)
    s = dot * lax.rsqrt(nt * ns)
    p = jnp.exp(s / NCE_T) / float(N_DATA)
    part = jnp.sum(jnp.log(MPN / (p + MPN + EPS)))

    @pl.when(i == 0)
    def _():
        s0_o[0, 0] = 0.0

    s0_o[0, 0] += part
    mt_out[...] = mt_in[...]
    ms_out[...] = ms_in[...]


def _slab_map(i):
    j = jnp.minimum(i, NSLAB - 1)
    return (j, 0, 0)


_c2 = pl.pallas_call(
    _c2_body,
    grid=(BATCH,),
    in_specs=[
        pl.BlockSpec((1, K, FEAT), lambda i: (i, 0, 0)),
        pl.BlockSpec((1, K, FEAT), lambda i: (i, 0, 0)),
        pl.BlockSpec((1, 1, FEAT), lambda i: (i, 0, 0)),
        pl.BlockSpec((1, 1, FEAT), lambda i: (i, 0, 0)),
        pl.BlockSpec((FEAT, FEAT), lambda i: (0, 0)),
        pl.BlockSpec((1, FEAT), lambda i: (0, 0)),
        pl.BlockSpec((1, SLAB, FEAT), _slab_map),
        pl.BlockSpec((1, SLAB, FEAT), _slab_map),
    ],
    out_specs=(
        pl.BlockSpec((1, 1), lambda i: (0, 0), memory_space=pltpu.SMEM),
        pl.BlockSpec((1, SLAB, FEAT), _slab_map),
        pl.BlockSpec((1, SLAB, FEAT), _slab_map),
    ),
    out_shape=(
        jax.ShapeDtypeStruct((1, 1), jnp.float32),
        jax.ShapeDtypeStruct((NSLAB, SLAB, FEAT), jnp.float32),
        jax.ShapeDtypeStruct((NSLAB, SLAB, FEAT), jnp.float32),
    ),
)


def _scat_body(idx_ref, upd_t, upd_s, mtc, msc, out_t, out_s, sem):
    def fire(b, carry):
        r = idx_ref[b]
        pltpu.async_copy(upd_t.at[b], out_t.at[r], sem)
        pltpu.async_copy(upd_s.at[b], out_s.at[r], sem)
        return carry

    lax.fori_loop(0, BATCH, fire, 0)

    def drain(b, carry):
        pltpu.make_async_copy(upd_t.at[0], out_t.at[0], sem).wait()
        pltpu.make_async_copy(upd_s.at[0], out_s.at[0], sem).wait()
        return carry

    lax.fori_loop(0, BATCH, drain, 0)


def _scatter2(mtc, msc, idx, upd_t, upd_s):
    grid_spec = pltpu.PrefetchScalarGridSpec(
        num_scalar_prefetch=1,
        grid=(1,),
        in_specs=[
            pl.BlockSpec((BATCH, FEAT), lambda i, idx: (0, 0)),
            pl.BlockSpec((BATCH, FEAT), lambda i, idx: (0, 0)),
            pl.BlockSpec(memory_space=pl.ANY),
            pl.BlockSpec(memory_space=pl.ANY),
        ],
        out_specs=(
            pl.BlockSpec(memory_space=pl.ANY),
            pl.BlockSpec(memory_space=pl.ANY),
        ),
        scratch_shapes=[pltpu.SemaphoreType.DMA],
    )
    return pl.pallas_call(
        _scat_body,
        grid_spec=grid_spec,
        out_shape=(
            jax.ShapeDtypeStruct((N_DATA, FEAT), jnp.float32),
            jax.ShapeDtypeStruct((N_DATA, FEAT), jnp.float32),
        ),
        input_output_aliases={3: 0, 4: 1},
    )(idx, upd_t, upd_s, mtc, msc)


def kernel(f_s, f_t, idx, contrast_idx, W_s, b_s, W_t, b_t,
           memory_s, memory_t, W_mt, b_mt):
    cidx = contrast_idx.reshape(NEG)
    wsp, wtp = _make_gather_pos()(memory_s, memory_t, idx)
    wsn, wtn = _make_gather()(memory_s, memory_t, cidx)
    b_s2 = b_s.reshape(1, FEAT)
    b_t2 = b_t.reshape(1, FEAT)
    b_mt2 = b_mt.reshape(1, FEAT)
    hs, ht, upd_s, upd_t, s1 = _c1(f_s, f_t, W_s, b_s2, W_t, b_t2,
                                   wsp, wtp, W_mt, b_mt2,
                                   idx.reshape(1, BATCH))
    s0, mt_copy, ms_copy = _c2(
        wsn.reshape(BATCH, K, FEAT), wtn.reshape(BATCH, K, FEAT),
        hs.reshape(BATCH, 1, FEAT), ht.reshape(BATCH, 1, FEAT),
        W_mt, b_mt2,
        memory_t.reshape(NSLAB, SLAB, FEAT), memory_s.reshape(NSLAB, SLAB, FEAT))
    loss = (-(s1[0, 0] + s0[0, 0]) / BATCH).reshape(1)
    new_mt, new_ms = _scatter2(mt_copy.reshape(N_DATA, FEAT),
                               ms_copy.reshape(N_DATA, FEAT),
                               idx, upd_t, upd_s)
    return loss, new_mt, new_ms
